# Initial kernel scaffold; baseline (speedup 1.0000x reference)
import functools

import jax
import jax.numpy as jnp
from jax import lax
from jax.experimental import pallas as pl
from jax.experimental.pallas import tpu as pltpu
from jax.experimental.pallas import tpu_sc as plsc

_NC = 2
_NS = 16
_NW = _NC * _NS
_LANES = 128
_G = 64
_STEPS = 2
_BN = 1000


def _trans_body(h_ref, w_ref, o_ref):
    o_ref[0] = jnp.dot(h_ref[...], w_ref[0],
                       preferred_element_type=jnp.float32,
                       precision=lax.Precision.HIGHEST)


def _edge_trans(h, w_edge):
    n, d = h.shape
    t = w_edge.shape[0]
    nb = n // _BN
    return pl.pallas_call(
        _trans_body,
        grid=(nb, t),
        in_specs=[
            pl.BlockSpec((_BN, d), lambda i, j: (i, 0)),
            pl.BlockSpec((1, d, d), lambda i, j: (j, 0, 0)),
        ],
        out_specs=pl.BlockSpec((1, _BN, d), lambda i, j: (j, i, 0)),
        out_shape=jax.ShapeDtypeStruct((t, n, d), jnp.float32),
    )(h, w_edge)


def _gru_body(m_ref, h_ref, wz, uz, bz, wr, ur, br, wh, uh, bh, o_ref):
    m = m_ref[0] + m_ref[1]
    hh = h_ref[...]
    dot = functools.partial(jnp.dot, preferred_element_type=jnp.float32,
                            precision=lax.Precision.HIGHEST)
    z = jax.nn.sigmoid(dot(m, wz[...]) + dot(hh, uz[...]) + bz[...])
    r = jax.nn.sigmoid(dot(m, wr[...]) + dot(hh, ur[...]) + br[...])
    ht = jnp.tanh(dot(m, wh[...]) + dot(r * hh, uh[...]) + bh[...])
    o_ref[...] = (1.0 - z) * hh + z * ht


def _gru(mparts, h, wz, uz, bz, wr, ur, br, wh, uh, bh):
    n, d = h.shape
    nb = n // _BN
    wspec = pl.BlockSpec((d, d), lambda i: (0, 0))
    bspec = pl.BlockSpec((1, d), lambda i: (0, 0))
    return pl.pallas_call(
        _gru_body,
        grid=(nb,),
        in_specs=[
            pl.BlockSpec((2, _BN, d), lambda i: (0, i, 0)),
            pl.BlockSpec((_BN, d), lambda i: (i, 0)),
            wspec, wspec, bspec, wspec, wspec, bspec, wspec, wspec, bspec,
        ],
        out_specs=pl.BlockSpec((_BN, d), lambda i: (i, 0)),
        out_shape=jax.ShapeDtypeStruct((n, d), jnp.float32),
    )(mparts, h, wz, uz, bz, wr, ur, br, wh, uh, bh)


def _seg_body(nchunk, nacc, nout, table, gidx_h, dst_h, out_h,
              idx_v, dst_v, buf_v, zero_v, acc, sem):
    c = lax.axis_index("c")
    s = lax.axis_index("s")
    wid = c * _NS + s
    d = zero_v.shape[1]
    zr = zero_v.shape[0]

    def _zrow(i, carry):
        for j in range(d // 16):
            zero_v[i, pl.ds(j * 16, 16)] = jnp.zeros((16,), jnp.float32)
        return carry
    lax.fori_loop(0, zr, _zrow, 0)

    rpt = nacc // _NS
    base = s * rpt
    for k in range(rpt // zr):
        pltpu.sync_copy(zero_v, acc.at[pl.ds(base + k * zr, zr)])
    plsc.subcore_barrier()

    pltpu.sync_copy(gidx_h.at[wid], idx_v)
    pltpu.sync_copy(dst_h.at[wid], dst_v)

    def _chunk(g, carry):
        pltpu.async_copy(table.at[idx_v.at[g]], buf_v, sem).wait()
        pltpu.sync_copy(buf_v, acc.at[dst_v.at[g]], add=True)
        return carry
    lax.fori_loop(0, nchunk, _chunk, 0)
    plsc.subcore_barrier()

    orow = nout // _NS
    pltpu.sync_copy(acc.at[pl.ds(s * orow, orow)],
                    out_h.at[c, pl.ds(s * orow, orow)])


def _sc_segsum(table, gidx_p, dst_p, nacc, nout):
    _, nchunk, lanes = gidx_p.shape
    d = table.shape[1]
    rpt = nacc // _NS
    zr = min(128, rpt)
    kern = pl.kernel(
        functools.partial(_seg_body, nchunk, nacc, nout),
        out_type=jax.ShapeDtypeStruct((_NC, nout, d), jnp.float32),
        mesh=plsc.VectorSubcoreMesh(core_axis_name="c", subcore_axis_name="s"),
        scratch_types=[
            pltpu.VMEM((nchunk, lanes), jnp.int32),
            pltpu.VMEM((nchunk, lanes), jnp.int32),
            pltpu.VMEM((lanes, d), jnp.float32),
            pltpu.VMEM((zr, d), jnp.float32),
            pltpu.VMEM_SHARED((nacc, d), jnp.float32),
            pltpu.SemaphoreType.DMA,
        ],
    )
    return kern(table, gidx_p, dst_p)


def _head_body(hs_ref, org_ref, w1, w2, b1, b2, o_ref):
    dot = functools.partial(jnp.dot, preferred_element_type=jnp.float32,
                            precision=lax.Precision.HIGHEST)
    sums = hs_ref[0] + hs_ref[1]
    w = dot(w1[...], w2[...])
    s = dot(sums, w)
    c = dot(b1[...], w2[...]) + b2[...]
    org = org_ref[...]
    gids = lax.broadcasted_iota(jnp.int32, (_G, org.shape[1]), 0)
    eq = org == gids
    counts = jnp.sum(jnp.where(eq, 1.0, 0.0), axis=1, keepdims=True)
    val = (s + counts * c) / jnp.maximum(counts, 1.0)
    o_ref[...] = jax.nn.sigmoid(val)


def _head_final(hsum, org2, w1, w2, b1, b2):
    d = w1.shape[0]
    h_count = org2.shape[1]
    return pl.pallas_call(
        _head_body,
        in_specs=[
            pl.BlockSpec((_NC, _G, d), lambda: (0, 0, 0)),
            pl.BlockSpec((1, h_count), lambda: (0, 0)),
            pl.BlockSpec((d, d), lambda: (0, 0)),
            pl.BlockSpec((d, 1), lambda: (0, 0)),
            pl.BlockSpec((1, d), lambda: (0, 0)),
            pl.BlockSpec((1, 1), lambda: (0, 0)),
        ],
        out_specs=pl.BlockSpec((_G, 1), lambda: (0, 0)),
        out_shape=jax.ShapeDtypeStruct((_G, 1), jnp.float32),
    )(hsum, org2, w1, w2, b1, b2)


def _pad_indices(vals, fill, nchunk):
    cap = _NW * nchunk * _LANES
    pad = cap - vals.shape[0]
    return jnp.concatenate(
        [vals, jnp.full((pad,), fill, jnp.int32)]).reshape(_NW, nchunk, _LANES)


def kernel(x, edge_index, edge_type, hole_locs, origins, W_edge,
           Wz, Uz, bz, Wr, Ur, br, Wh, Uh, bh, W1, b1, W2, b2):
    n, d = x.shape
    e = edge_index.shape[1]
    t = W_edge.shape[0]
    h_count = hole_locs.shape[0]

    src = edge_index[0]
    dst = edge_index[1]
    gidx = edge_type * n + src

    nchunk = 2 * (-(-e // (_NW * _LANES * 2)))
    gidx_p = _pad_indices(gidx, 0, nchunk)
    dst_p = _pad_indices(dst, n, nchunk)

    nacc = (-(-(n + 1) // (_NS * _LANES))) * (_NS * _LANES)

    hchunk = -(-h_count // (_NW * _LANES))
    hol_p = _pad_indices(hole_locs, 0, hchunk)
    org_p = _pad_indices(origins, _G, hchunk)
    hacc = (-(-(_G + 1) // _NS)) * _NS

    bzr, brr, bhr = bz.reshape(1, d), br.reshape(1, d), bh.reshape(1, d)
    b1r, b2r = b1.reshape(1, d), b2.reshape(1, 1)

    h = x
    for _ in range(_STEPS):
        at = _edge_trans(h, W_edge)
        table = at.reshape(t * n, d)
        mparts = _sc_segsum(table, gidx_p, dst_p, nacc, n)
        h = _gru(mparts, h, Wz, Uz, bzr, Wr, Ur, brr, Wh, Uh, bhr)

    hsum = _sc_segsum(h, hol_p, org_p, hacc, _G)
    org2 = origins.reshape(1, h_count)
    preds = _head_final(hsum, org2, W1, W2, b1r, b2r)
    return preds.reshape(_G)


# speedup vs baseline: 9.2765x; 9.2765x over previous
import functools

import jax
import jax.numpy as jnp
from jax import lax
from jax.experimental import pallas as pl
from jax.experimental.pallas import tpu as pltpu
from jax.experimental.pallas import tpu_sc as plsc

_NC = 2
_NS = 16
_NW = _NC * _NS
_LANES = 128
_G = 64
_STEPS = 2
_BN = 1000


def _trans_body(h_ref, w_ref, o_ref):
    o_ref[0] = jnp.dot(h_ref[...], w_ref[0],
                       preferred_element_type=jnp.float32,
                       precision=lax.Precision.HIGHEST)


def _edge_trans(h, w_edge):
    n, d = h.shape
    t = w_edge.shape[0]
    nb = n // _BN
    return pl.pallas_call(
        _trans_body,
        grid=(nb, t),
        in_specs=[
            pl.BlockSpec((_BN, d), lambda i, j: (i, 0)),
            pl.BlockSpec((1, d, d), lambda i, j: (j, 0, 0)),
        ],
        out_specs=pl.BlockSpec((1, _BN, d), lambda i, j: (j, i, 0)),
        out_shape=jax.ShapeDtypeStruct((t, n, d), jnp.float32),
    )(h, w_edge)


def _gru_body(m_ref, h_ref, wz, uz, bz, wr, ur, br, wh, uh, bh, o_ref):
    m = m_ref[0] + m_ref[1]
    hh = h_ref[...]
    dot = functools.partial(jnp.dot, preferred_element_type=jnp.float32,
                            precision=lax.Precision.HIGHEST)
    z = jax.nn.sigmoid(dot(m, wz[...]) + dot(hh, uz[...]) + bz[...])
    r = jax.nn.sigmoid(dot(m, wr[...]) + dot(hh, ur[...]) + br[...])
    ht = jnp.tanh(dot(m, wh[...]) + dot(r * hh, uh[...]) + bh[...])
    o_ref[...] = (1.0 - z) * hh + z * ht


def _gru(mparts, h, wz, uz, bz, wr, ur, br, wh, uh, bh):
    n, d = h.shape
    nb = n // _BN
    wspec = pl.BlockSpec((d, d), lambda i: (0, 0))
    bspec = pl.BlockSpec((1, d), lambda i: (0, 0))
    return pl.pallas_call(
        _gru_body,
        grid=(nb,),
        in_specs=[
            pl.BlockSpec((2, _BN, d), lambda i: (0, i, 0)),
            pl.BlockSpec((_BN, d), lambda i: (i, 0)),
            wspec, wspec, bspec, wspec, wspec, bspec, wspec, wspec, bspec,
        ],
        out_specs=pl.BlockSpec((_BN, d), lambda i: (i, 0)),
        out_shape=jax.ShapeDtypeStruct((n, d), jnp.float32),
    )(mparts, h, wz, uz, bz, wr, ur, br, wh, uh, bh)


def _seg_body(nchunk, nacc, table, gidx_h, dst_h, out_h,
              idx_v, dst_v, buf_v, zero_v, acc, sem):
    c = lax.axis_index("c")
    s = lax.axis_index("s")
    wid = c * _NS + s
    d = zero_v.shape[1]
    zr = zero_v.shape[0]

    def _zrow(i, carry):
        for j in range(d // 16):
            zero_v[i, pl.ds(j * 16, 16)] = jnp.zeros((16,), jnp.float32)
        return carry
    lax.fori_loop(0, zr, _zrow, 0)

    rpt = nacc // _NS
    base = s * rpt

    def _zcopy(k, carry):
        pltpu.sync_copy(zero_v, acc.at[pl.ds(base + k * zr, zr)])
        return carry
    lax.fori_loop(0, rpt // zr, _zcopy, 0)
    plsc.subcore_barrier()

    pltpu.sync_copy(gidx_h.at[wid], idx_v)
    pltpu.sync_copy(dst_h.at[wid], dst_v)

    def _chunk(g, carry):
        pltpu.async_copy(table.at[idx_v.at[g]], buf_v, sem).wait()
        pltpu.sync_copy(buf_v, acc.at[dst_v.at[g]], add=True)
        return carry
    lax.fori_loop(0, nchunk, _chunk, 0)
    plsc.subcore_barrier()

    pltpu.sync_copy(acc.at[pl.ds(base, rpt)],
                    out_h.at[c, pl.ds(base, rpt)])


def _sc_segsum(table, gidx_p, dst_p, nacc):
    _, nchunk, lanes = gidx_p.shape
    d = table.shape[1]
    zr = 8
    kern = pl.kernel(
        functools.partial(_seg_body, nchunk, nacc),
        out_type=jax.ShapeDtypeStruct((_NC, nacc, d), jnp.float32),
        mesh=plsc.VectorSubcoreMesh(core_axis_name="c", subcore_axis_name="s"),
        scratch_types=[
            pltpu.VMEM((nchunk, lanes), jnp.int32),
            pltpu.VMEM((nchunk, lanes), jnp.int32),
            pltpu.VMEM((lanes, d), jnp.float32),
            pltpu.VMEM((zr, d), jnp.float32),
            pltpu.VMEM_SHARED((nacc, d), jnp.float32),
            pltpu.SemaphoreType.DMA,
        ],
    )
    return kern(table, gidx_p, dst_p)


def _head_body(hs_ref, org_ref, w1, w2, b1, b2, o_ref):
    dot = functools.partial(jnp.dot, preferred_element_type=jnp.float32,
                            precision=lax.Precision.HIGHEST)
    sums = hs_ref[0, :_G] + hs_ref[1, :_G]
    w = dot(w1[...], w2[...])
    s = dot(sums, w)
    c = dot(b1[...], w2[...]) + b2[...]
    org = org_ref[...]
    gids = lax.broadcasted_iota(jnp.int32, (_G, org.shape[1]), 0)
    eq = org == gids
    counts = jnp.sum(jnp.where(eq, 1.0, 0.0), axis=1, keepdims=True)
    val = (s + counts * c) / jnp.maximum(counts, 1.0)
    o_ref[...] = jax.nn.sigmoid(val)


def _head_final(hsum, org2, w1, w2, b1, b2):
    return pl.pallas_call(
        _head_body,
        out_shape=jax.ShapeDtypeStruct((_G, 1), jnp.float32),
    )(hsum, org2, w1, w2, b1, b2)


def _pad_indices(vals, fill, nchunk):
    cap = _NW * nchunk * _LANES
    pad = cap - vals.shape[0]
    return jnp.concatenate(
        [vals, jnp.full((pad,), fill, jnp.int32)]).reshape(_NW, nchunk, _LANES)


def kernel(x, edge_index, edge_type, hole_locs, origins, W_edge,
           Wz, Uz, bz, Wr, Ur, br, Wh, Uh, bh, W1, b1, W2, b2):
    n, d = x.shape
    e = edge_index.shape[1]
    t = W_edge.shape[0]
    h_count = hole_locs.shape[0]

    src = edge_index[0]
    dst = edge_index[1]
    gidx = edge_type * n + src

    nchunk = 2 * (-(-e // (_NW * _LANES * 2)))
    gidx_p = _pad_indices(gidx, 0, nchunk)
    dst_p = _pad_indices(dst, n, nchunk)

    nacc = (-(-(n + 1) // _LANES)) * _LANES

    hchunk = -(-h_count // (_NW * _LANES))
    hol_p = _pad_indices(hole_locs, 0, hchunk)
    org_p = _pad_indices(origins, _G, hchunk)
    hacc = (-(-(_G + 1) // (_NS * 8))) * (_NS * 8)

    bzr, brr, bhr = bz.reshape(1, d), br.reshape(1, d), bh.reshape(1, d)
    b1r, b2r = b1.reshape(1, d), b2.reshape(1, 1)

    h = x
    for _ in range(_STEPS):
        at = _edge_trans(h, W_edge)
        table = at.reshape(t * n, d)
        mparts = _sc_segsum(table, gidx_p, dst_p, nacc)
        h = _gru(mparts, h, Wz, Uz, bzr, Wr, Ur, brr, Wh, Uh, bhr)

    hsum = _sc_segsum(h, hol_p, org_p, hacc)
    org2 = origins.reshape(1, h_count)
    preds = _head_final(hsum, org2, W1, W2, b1r, b2r)
    return preds.reshape(_G)


# speedup vs baseline: 21.6672x; 2.3357x over previous
import functools

import jax
import jax.numpy as jnp
from jax import lax
from jax.experimental import pallas as pl
from jax.experimental.pallas import tpu as pltpu
from jax.experimental.pallas import tpu_sc as plsc

_NC = 2
_NS = 16
_NW = _NC * _NS
_LANES = 128
_G = 64
_STEPS = 2
_BN = 1000


def _trans_body(h_ref, w_ref, o_ref):
    o_ref[0] = jnp.dot(h_ref[...], w_ref[0],
                       preferred_element_type=jnp.float32,
                       precision=lax.Precision.HIGHEST)


def _edge_trans(h, w_edge):
    n, d = h.shape
    t = w_edge.shape[0]
    nb = n // _BN
    return pl.pallas_call(
        _trans_body,
        grid=(nb, t),
        in_specs=[
            pl.BlockSpec((_BN, d), lambda i, j: (i, 0)),
            pl.BlockSpec((1, d, d), lambda i, j: (j, 0, 0)),
        ],
        out_specs=pl.BlockSpec((1, _BN, d), lambda i, j: (j, i, 0)),
        out_shape=jax.ShapeDtypeStruct((t, n, d), jnp.float32),
    )(h, w_edge)


def _gru_body(m_ref, h_ref, wz, uz, bz, wr, ur, br, wh, uh, bh, o_ref):
    m = m_ref[0] + m_ref[1]
    hh = h_ref[...]
    dot = functools.partial(jnp.dot, preferred_element_type=jnp.float32,
                            precision=lax.Precision.HIGHEST)
    z = jax.nn.sigmoid(dot(m, wz[...]) + dot(hh, uz[...]) + bz[...])
    r = jax.nn.sigmoid(dot(m, wr[...]) + dot(hh, ur[...]) + br[...])
    ht = jnp.tanh(dot(m, wh[...]) + dot(r * hh, uh[...]) + bh[...])
    o_ref[...] = (1.0 - z) * hh + z * ht


def _gru(mparts, h, wz, uz, bz, wr, ur, br, wh, uh, bh):
    n, d = h.shape
    nb = n // _BN
    wspec = pl.BlockSpec((d, d), lambda i: (0, 0))
    bspec = pl.BlockSpec((1, d), lambda i: (0, 0))
    return pl.pallas_call(
        _gru_body,
        grid=(nb,),
        in_specs=[
            pl.BlockSpec((2, _BN, d), lambda i: (0, i, 0)),
            pl.BlockSpec((_BN, d), lambda i: (i, 0)),
            wspec, wspec, bspec, wspec, wspec, bspec, wspec, wspec, bspec,
        ],
        out_specs=pl.BlockSpec((_BN, d), lambda i: (i, 0)),
        out_shape=jax.ShapeDtypeStruct((n, d), jnp.float32),
    )(mparts, h, wz, uz, bz, wr, ur, br, wh, uh, bh)


def _seg_body(nchunk, nacc, table, gidx_h, dst_h, out_h,
              idx_v, dst_v, buf_v, zero_v, acc, sem):
    c = lax.axis_index("c")
    s = lax.axis_index("s")
    wid = c * _NS + s
    d = zero_v.shape[1]
    zr = zero_v.shape[0]

    def _zrow(i, carry):
        for j in range(d // 16):
            zero_v[i, pl.ds(j * 16, 16)] = jnp.zeros((16,), jnp.float32)
        return carry
    lax.fori_loop(0, zr, _zrow, 0)

    rpt = nacc // _NS
    base = s * rpt

    def _zcopy(k, carry):
        pltpu.sync_copy(zero_v, acc.at[pl.ds(base + k * zr, zr)])
        return carry
    lax.fori_loop(0, rpt // zr, _zcopy, 0)
    plsc.subcore_barrier()

    pltpu.sync_copy(gidx_h.at[wid], idx_v)
    pltpu.sync_copy(dst_h.at[wid], dst_v)

    def _chunk(g, carry):
        pltpu.async_copy(table.at[idx_v.at[g]], buf_v, sem).wait()
        pltpu.sync_copy(buf_v, acc.at[dst_v.at[g]], add=True)
        return carry
    lax.fori_loop(0, nchunk, _chunk, 0)
    plsc.subcore_barrier()

    pltpu.sync_copy(acc.at[pl.ds(base, rpt)],
                    out_h.at[c, pl.ds(base, rpt)])


def _sc_segsum(table, gidx_p, dst_p, nacc):
    _, nchunk, lanes = gidx_p.shape
    d = table.shape[1]
    zr = 8
    kern = pl.kernel(
        functools.partial(_seg_body, nchunk, nacc),
        out_type=jax.ShapeDtypeStruct((_NC, nacc, d), jnp.float32),
        mesh=plsc.VectorSubcoreMesh(core_axis_name="c", subcore_axis_name="s"),
        scratch_types=[
            pltpu.VMEM((nchunk, lanes), jnp.int32),
            pltpu.VMEM((nchunk, lanes), jnp.int32),
            pltpu.VMEM((lanes, d), jnp.float32),
            pltpu.VMEM((zr, d), jnp.float32),
            pltpu.VMEM_SHARED((nacc, d), jnp.float32),
            pltpu.SemaphoreType.DMA,
        ],
    )
    return kern(table, gidx_p, dst_p)


def _head_body(hs_ref, org_ref, w1, w2, b1, b2, o_ref):
    dot = functools.partial(jnp.dot, preferred_element_type=jnp.float32,
                            precision=lax.Precision.HIGHEST)
    sums = hs_ref[0, :_G] + hs_ref[1, :_G]
    w = dot(w1[...], w2[...])
    s = dot(sums, w)
    c = dot(b1[...], w2[...]) + b2[...]
    org = org_ref[...]
    gids = lax.broadcasted_iota(jnp.int32, (_G, org.shape[1]), 0)
    eq = org == gids
    counts = jnp.sum(jnp.where(eq, 1.0, 0.0), axis=1, keepdims=True)
    val = (s + counts * c) / jnp.maximum(counts, 1.0)
    o_ref[...] = jax.nn.sigmoid(val)


def _head_final(hsum, org2, w1, w2, b1, b2):
    return pl.pallas_call(
        _head_body,
        out_shape=jax.ShapeDtypeStruct((_G, 1), jnp.float32),
    )(hsum, org2, w1, w2, b1, b2)


def _pad_indices(vals, fill_base, fill_mod, nchunk):
    cap = _NW * nchunk * _LANES
    pad = cap - vals.shape[0]
    filler = fill_base + jnp.arange(pad, dtype=jnp.int32) % fill_mod
    return jnp.concatenate([vals, filler]).reshape(_NW, nchunk, _LANES)


def kernel(x, edge_index, edge_type, hole_locs, origins, W_edge,
           Wz, Uz, bz, Wr, Ur, br, Wh, Uh, bh, W1, b1, W2, b2):
    n, d = x.shape
    e = edge_index.shape[1]
    t = W_edge.shape[0]
    h_count = hole_locs.shape[0]

    src = edge_index[0]
    dst = edge_index[1]
    gidx = edge_type * n + src

    nacc = (-(-(n + 1) // _LANES)) * _LANES
    nchunk = 2 * (-(-e // (_NW * _LANES * 2)))
    gidx_p = _pad_indices(gidx, 0, t * n, nchunk)
    dst_p = _pad_indices(dst, n, nacc - n, nchunk)

    hacc = (-(-(_G + 1) // (_NS * 8))) * (_NS * 8)
    hchunk = -(-h_count // (_NW * _LANES))
    hol_p = _pad_indices(hole_locs, 0, n, hchunk)
    org_p = _pad_indices(origins, _G, hacc - _G, hchunk)

    bzr, brr, bhr = bz.reshape(1, d), br.reshape(1, d), bh.reshape(1, d)
    b1r, b2r = b1.reshape(1, d), b2.reshape(1, 1)

    h = x
    for _ in range(_STEPS):
        at = _edge_trans(h, W_edge)
        table = at.reshape(t * n, d)
        mparts = _sc_segsum(table, gidx_p, dst_p, nacc)
        h = _gru(mparts, h, Wz, Uz, bzr, Wr, Ur, brr, Wh, Uh, bhr)

    hsum = _sc_segsum(h, hol_p, org_p, hacc)
    org2 = origins.reshape(1, h_count)
    preds = _head_final(hsum, org2, W1, W2, b1r, b2r)
    return preds.reshape(_G)


# speedup vs baseline: 26.2956x; 1.2136x over previous
import functools

import jax
import jax.numpy as jnp
from jax import lax
from jax.experimental import pallas as pl
from jax.experimental.pallas import tpu as pltpu
from jax.experimental.pallas import tpu_sc as plsc

_NC = 2
_NS = 16
_NW = _NC * _NS
_LANES = 128
_G = 64
_STEPS = 2
_BN = 1000


def _trans_body(h_ref, w_ref, o_ref):
    o_ref[0] = jnp.dot(h_ref[...], w_ref[0],
                       preferred_element_type=jnp.float32)


def _edge_trans(h, w_edge):
    n, d = h.shape
    t = w_edge.shape[0]
    nb = n // _BN
    return pl.pallas_call(
        _trans_body,
        grid=(nb, t),
        in_specs=[
            pl.BlockSpec((_BN, d), lambda i, j: (i, 0)),
            pl.BlockSpec((1, d, d), lambda i, j: (j, 0, 0)),
        ],
        out_specs=pl.BlockSpec((1, _BN, d), lambda i, j: (j, i, 0)),
        out_shape=jax.ShapeDtypeStruct((t, n, d), jnp.float32),
    )(h, w_edge)


def _gru_body(m_ref, h_ref, wz, uz, bz, wr, ur, br, wh, uh, bh, o_ref):
    m = m_ref[0] + m_ref[1]
    hh = h_ref[...]
    dot = functools.partial(jnp.dot, preferred_element_type=jnp.float32)
    z = jax.nn.sigmoid(dot(m, wz[...]) + dot(hh, uz[...]) + bz[...])
    r = jax.nn.sigmoid(dot(m, wr[...]) + dot(hh, ur[...]) + br[...])
    ht = jnp.tanh(dot(m, wh[...]) + dot(r * hh, uh[...]) + bh[...])
    o_ref[...] = (1.0 - z) * hh + z * ht


def _gru(mparts, h, wz, uz, bz, wr, ur, br, wh, uh, bh):
    n, d = h.shape
    nb = n // _BN
    wspec = pl.BlockSpec((d, d), lambda i: (0, 0))
    bspec = pl.BlockSpec((1, d), lambda i: (0, 0))
    return pl.pallas_call(
        _gru_body,
        grid=(nb,),
        in_specs=[
            pl.BlockSpec((2, _BN, d), lambda i: (0, i, 0)),
            pl.BlockSpec((_BN, d), lambda i: (i, 0)),
            wspec, wspec, bspec, wspec, wspec, bspec, wspec, wspec, bspec,
        ],
        out_specs=pl.BlockSpec((_BN, d), lambda i: (i, 0)),
        out_shape=jax.ShapeDtypeStruct((n, d), jnp.float32),
    )(mparts, h, wz, uz, bz, wr, ur, br, wh, uh, bh)


def _seg_body(nchunk, nacc, table, gidx_h, dst_h, out_h,
              idx_v, dst_v, buf_v, zero_v, acc, sem):
    c = lax.axis_index("c")
    s = lax.axis_index("s")
    wid = c * _NS + s
    d = zero_v.shape[1]
    zr = zero_v.shape[0]

    def _zrow(i, carry):
        for j in range(d // 16):
            zero_v[i, pl.ds(j * 16, 16)] = jnp.zeros((16,), jnp.float32)
        return carry
    lax.fori_loop(0, zr, _zrow, 0)

    rpt = nacc // _NS
    base = s * rpt

    def _zcopy(k, carry):
        pltpu.sync_copy(zero_v, acc.at[pl.ds(base + k * zr, zr)])
        return carry
    lax.fori_loop(0, rpt // zr, _zcopy, 0)
    plsc.subcore_barrier()

    pltpu.sync_copy(gidx_h.at[wid], idx_v)
    pltpu.sync_copy(dst_h.at[wid], dst_v)

    def _chunk(g, carry):
        pltpu.async_copy(table.at[idx_v.at[g]], buf_v, sem).wait()
        pltpu.sync_copy(buf_v, acc.at[dst_v.at[g]], add=True)
        return carry
    lax.fori_loop(0, nchunk, _chunk, 0)
    plsc.subcore_barrier()

    pltpu.sync_copy(acc.at[pl.ds(base, rpt)],
                    out_h.at[c, pl.ds(base, rpt)])


def _sc_segsum(table, gidx_p, dst_p, nacc):
    _, nchunk, lanes = gidx_p.shape
    d = table.shape[1]
    zr = 8
    kern = pl.kernel(
        functools.partial(_seg_body, nchunk, nacc),
        out_type=jax.ShapeDtypeStruct((_NC, nacc, d), jnp.float32),
        mesh=plsc.VectorSubcoreMesh(core_axis_name="c", subcore_axis_name="s"),
        scratch_types=[
            pltpu.VMEM((nchunk, lanes), jnp.int32),
            pltpu.VMEM((nchunk, lanes), jnp.int32),
            pltpu.VMEM((lanes, d), jnp.float32),
            pltpu.VMEM((zr, d), jnp.float32),
            pltpu.VMEM_SHARED((nacc, d), jnp.float32),
            pltpu.SemaphoreType.DMA,
        ],
    )
    return kern(table, gidx_p, dst_p)


def _head_body(hs_ref, org_ref, w1, w2, b1, b2, o_ref):
    dot = functools.partial(jnp.dot, preferred_element_type=jnp.float32)
    sums = hs_ref[0, :_G] + hs_ref[1, :_G]
    w = dot(w1[...], w2[...])
    s = dot(sums, w)
    c = dot(b1[...], w2[...]) + b2[...]
    org = org_ref[...]
    gids = lax.broadcasted_iota(jnp.int32, (_G, org.shape[1]), 0)
    eq = org == gids
    counts = jnp.sum(jnp.where(eq, 1.0, 0.0), axis=1, keepdims=True)
    val = (s + counts * c) / jnp.maximum(counts, 1.0)
    o_ref[...] = jax.nn.sigmoid(val)


def _head_final(hsum, org2, w1, w2, b1, b2):
    return pl.pallas_call(
        _head_body,
        out_shape=jax.ShapeDtypeStruct((_G, 1), jnp.float32),
    )(hsum, org2, w1, w2, b1, b2)


def _pad_indices(vals, fill_base, fill_mod, nchunk):
    cap = _NW * nchunk * _LANES
    pad = cap - vals.shape[0]
    filler = fill_base + jnp.arange(pad, dtype=jnp.int32) % fill_mod
    return jnp.concatenate([vals, filler]).reshape(_NW, nchunk, _LANES)


def kernel(x, edge_index, edge_type, hole_locs, origins, W_edge,
           Wz, Uz, bz, Wr, Ur, br, Wh, Uh, bh, W1, b1, W2, b2):
    n, d = x.shape
    e = edge_index.shape[1]
    t = W_edge.shape[0]
    h_count = hole_locs.shape[0]

    src = edge_index[0]
    dst = edge_index[1]
    gidx = edge_type * n + src

    nacc = (-(-(n + 1) // _LANES)) * _LANES
    nchunk = 2 * (-(-e // (_NW * _LANES * 2)))
    gidx_p = _pad_indices(gidx, 0, t * n, nchunk)
    dst_p = _pad_indices(dst, n, nacc - n, nchunk)

    hacc = (-(-(_G + 1) // (_NS * 8))) * (_NS * 8)
    hchunk = -(-h_count // (_NW * _LANES))
    hol_p = _pad_indices(hole_locs, 0, n, hchunk)
    org_p = _pad_indices(origins, _G, hacc - _G, hchunk)

    bzr, brr, bhr = bz.reshape(1, d), br.reshape(1, d), bh.reshape(1, d)
    b1r, b2r = b1.reshape(1, d), b2.reshape(1, 1)

    h = x
    for _ in range(_STEPS):
        at = _edge_trans(h, W_edge)
        table = at.reshape(t * n, d)
        mparts = _sc_segsum(table, gidx_p, dst_p, nacc)
        h = _gru(mparts, h, Wz, Uz, bzr, Wr, Ur, brr, Wh, Uh, bhr)

    hsum = _sc_segsum(h, hol_p, org_p, hacc)
    org2 = origins.reshape(1, h_count)
    preds = _head_final(hsum, org2, W1, W2, b1r, b2r)
    return preds.reshape(_G)


# speedup vs baseline: 36.1659x; 1.3754x over previous
import functools

import jax
import jax.numpy as jnp
from jax import lax
from jax.experimental import pallas as pl
from jax.experimental.pallas import tpu as pltpu
from jax.experimental.pallas import tpu_sc as plsc

_NC = 2
_NS = 16
_NW = _NC * _NS
_LANES = 128
_G = 64
_STEPS = 2
_BN = 1000


def _trans_body(h_ref, w_ref, o_ref):
    o_ref[0] = jnp.dot(h_ref[...], w_ref[0],
                       preferred_element_type=jnp.float32)


def _edge_trans(h, w_edge):
    n, d = h.shape
    t = w_edge.shape[0]
    nb = n // _BN
    return pl.pallas_call(
        _trans_body,
        grid=(nb, t),
        in_specs=[
            pl.BlockSpec((_BN, d), lambda i, j: (i, 0)),
            pl.BlockSpec((1, d, d), lambda i, j: (j, 0, 0)),
        ],
        out_specs=pl.BlockSpec((1, _BN, d), lambda i, j: (j, i, 0)),
        out_shape=jax.ShapeDtypeStruct((t, n, d), jnp.float32),
    )(h, w_edge)


def _gru_body(m_ref, h_ref, wz, uz, bz, wr, ur, br, wh, uh, bh, o_ref):
    m = m_ref[0] + m_ref[1]
    hh = h_ref[...]
    dot = functools.partial(jnp.dot, preferred_element_type=jnp.float32)
    z = jax.nn.sigmoid(dot(m, wz[...]) + dot(hh, uz[...]) + bz[...])
    r = jax.nn.sigmoid(dot(m, wr[...]) + dot(hh, ur[...]) + br[...])
    ht = jnp.tanh(dot(m, wh[...]) + dot(r * hh, uh[...]) + bh[...])
    o_ref[...] = (1.0 - z) * hh + z * ht


def _gru(mparts, h, wz, uz, bz, wr, ur, br, wh, uh, bh):
    n, d = h.shape
    nb = n // _BN
    wspec = pl.BlockSpec((d, d), lambda i: (0, 0))
    bspec = pl.BlockSpec((1, d), lambda i: (0, 0))
    return pl.pallas_call(
        _gru_body,
        grid=(nb,),
        in_specs=[
            pl.BlockSpec((2, _BN, d), lambda i: (0, i, 0)),
            pl.BlockSpec((_BN, d), lambda i: (i, 0)),
            wspec, wspec, bspec, wspec, wspec, bspec, wspec, wspec, bspec,
        ],
        out_specs=pl.BlockSpec((_BN, d), lambda i: (i, 0)),
        out_shape=jax.ShapeDtypeStruct((n, d), jnp.float32),
    )(mparts, h, wz, uz, bz, wr, ur, br, wh, uh, bh)


def _seg_body(nchunk, nacc, table, gidx_h, dst_h, out_h,
              idx_v, dst_v, buf_v, zero_v, acc, sem, sem2):
    c = lax.axis_index("c")
    s = lax.axis_index("s")
    wid = c * _NS + s
    d = zero_v.shape[1]
    zr = zero_v.shape[0]

    def _zrow(i, carry):
        for j in range(d // 16):
            zero_v[i, pl.ds(j * 16, 16)] = jnp.zeros((16,), jnp.float32)
        return carry
    lax.fori_loop(0, zr, _zrow, 0)

    rpt = nacc // _NS
    base = s * rpt

    def _zcopy(k, carry):
        pltpu.sync_copy(zero_v, acc.at[pl.ds(base + k * zr, zr)])
        return carry
    lax.fori_loop(0, rpt // zr, _zcopy, 0)
    plsc.subcore_barrier()

    nh = idx_v.shape[0]
    npair = nh // 2
    for half in range(nchunk // nh):
        pltpu.sync_copy(gidx_h.at[wid, pl.ds(half * nh, nh)], idx_v)
        pltpu.sync_copy(dst_h.at[wid, pl.ds(half * nh, nh)], dst_v)
        pltpu.async_copy(table.at[idx_v.at[0]], buf_v.at[0], sem)
        pltpu.async_copy(table.at[idx_v.at[1]], buf_v.at[1], sem2)

        def _pair(p, carry):
            pltpu.make_async_copy(table.at[idx_v.at[2 * p]],
                                  buf_v.at[0], sem).wait()
            pltpu.sync_copy(buf_v.at[0], acc.at[dst_v.at[2 * p]], add=True)

            @pl.when(p + 1 < npair)
            def _():
                pltpu.async_copy(table.at[idx_v.at[2 * p + 2]],
                                 buf_v.at[0], sem)
            pltpu.make_async_copy(table.at[idx_v.at[2 * p + 1]],
                                  buf_v.at[1], sem2).wait()
            pltpu.sync_copy(buf_v.at[1], acc.at[dst_v.at[2 * p + 1]],
                            add=True)

            @pl.when(p + 1 < npair)
            def _():
                pltpu.async_copy(table.at[idx_v.at[2 * p + 3]],
                                 buf_v.at[1], sem2)
            return carry
        lax.fori_loop(0, npair, _pair, 0)
    plsc.subcore_barrier()

    pltpu.sync_copy(acc.at[pl.ds(base, rpt)],
                    out_h.at[c, pl.ds(base, rpt)])


def _sc_segsum(table, gidx_p, dst_p, nacc):
    _, nchunk, lanes = gidx_p.shape
    d = table.shape[1]
    zr = 8
    kern = pl.kernel(
        functools.partial(_seg_body, nchunk, nacc),
        out_type=jax.ShapeDtypeStruct((_NC, nacc, d), jnp.float32),
        mesh=plsc.VectorSubcoreMesh(core_axis_name="c", subcore_axis_name="s"),
        scratch_types=[
            pltpu.VMEM((max(nchunk // 2, 2), lanes), jnp.int32),
            pltpu.VMEM((max(nchunk // 2, 2), lanes), jnp.int32),
            pltpu.VMEM((2, lanes, d), jnp.float32),
            pltpu.VMEM((zr, d), jnp.float32),
            pltpu.VMEM_SHARED((nacc, d), jnp.float32),
            pltpu.SemaphoreType.DMA,
            pltpu.SemaphoreType.DMA,
        ],
    )
    return kern(table, gidx_p, dst_p)


def _head_body(hs_ref, org_ref, w1, w2, b1, b2, o_ref):
    dot = functools.partial(jnp.dot, preferred_element_type=jnp.float32)
    sums = hs_ref[0, :_G] + hs_ref[1, :_G]
    w = dot(w1[...], w2[...])
    s = dot(sums, w)
    c = dot(b1[...], w2[...]) + b2[...]
    org = org_ref[...]
    gids = lax.broadcasted_iota(jnp.int32, (_G, org.shape[1]), 0)
    eq = org == gids
    counts = jnp.sum(jnp.where(eq, 1.0, 0.0), axis=1, keepdims=True)
    val = (s + counts * c) / jnp.maximum(counts, 1.0)
    o_ref[...] = jax.nn.sigmoid(val)


def _head_final(hsum, org2, w1, w2, b1, b2):
    return pl.pallas_call(
        _head_body,
        out_shape=jax.ShapeDtypeStruct((_G, 1), jnp.float32),
    )(hsum, org2, w1, w2, b1, b2)


def _pad_indices(vals, fill_base, fill_mod, nchunk):
    cap = _NW * nchunk * _LANES
    pad = cap - vals.shape[0]
    filler = fill_base + jnp.arange(pad, dtype=jnp.int32) % fill_mod
    return jnp.concatenate([vals, filler]).reshape(_NW, nchunk, _LANES)


def kernel(x, edge_index, edge_type, hole_locs, origins, W_edge,
           Wz, Uz, bz, Wr, Ur, br, Wh, Uh, bh, W1, b1, W2, b2):
    n, d = x.shape
    e = edge_index.shape[1]
    t = W_edge.shape[0]
    h_count = hole_locs.shape[0]

    src = edge_index[0]
    dst = edge_index[1]
    gidx = edge_type * n + src

    nacc = (-(-(n + 1) // _LANES)) * _LANES
    nchunk = 2 * (-(-e // (_NW * _LANES * 2)))
    gidx_p = _pad_indices(gidx, 0, t * n, nchunk)
    dst_p = _pad_indices(dst, n, nacc - n, nchunk)

    hacc = (-(-(_G + 1) // (_NS * 8))) * (_NS * 8)
    hchunk = -(-h_count // (_NW * _LANES))
    hol_p = _pad_indices(hole_locs, 0, n, hchunk)
    org_p = _pad_indices(origins, _G, hacc - _G, hchunk)

    bzr, brr, bhr = bz.reshape(1, d), br.reshape(1, d), bh.reshape(1, d)
    b1r, b2r = b1.reshape(1, d), b2.reshape(1, 1)

    h = x
    for _ in range(_STEPS):
        at = _edge_trans(h, W_edge)
        table = at.reshape(t * n, d)
        mparts = _sc_segsum(table, gidx_p, dst_p, nacc)
        h = _gru(mparts, h, Wz, Uz, bzr, Wr, Ur, brr, Wh, Uh, bhr)

    hsum = _sc_segsum(h, hol_p, org_p, hacc)
    org2 = origins.reshape(1, h_count)
    preds = _head_final(hsum, org2, W1, W2, b1r, b2r)
    return preds.reshape(_G)
